# baseline (speedup 1.0000x reference)
import jax
import jax.numpy as jnp
from jax import lax
from jax.experimental import pallas as pl
from jax.experimental.pallas import tpu as pltpu

F = 512
HID = 256
NC = 16
BUF = 256
K = 4
B = 32
NI = 1024
N = B + BUF

_HI = lax.Precision.HIGHEST


def _hi_dot(a, b):
    return lax.dot_general(a, b, (((1,), (0,)), ((), ())), precision=_HI,
                           preferred_element_type=jnp.float32)


def _hi_dot_t(a, b):
    return lax.dot_general(a, b, (((0,), (0,)), ((), ())), precision=_HI,
                           preferred_element_type=jnp.float32)


def _lrelu(x, slope):
    return jnp.where(x >= 0, x, slope * x)


def _attn_kernel(x_ref, aW1_ref, ab1_ref, aW2_ref, ab2_ref, M_ref):
    xb = x_ref[0]
    H = jnp.maximum(jnp.dot(xb, aW1_ref[...]) + ab1_ref[...], 0.0)
    a = jnp.dot(H, aW2_ref[...]) + ab2_ref[...]
    amax = jnp.max(a, axis=0, keepdims=True)
    e = jnp.exp(a - amax)
    w = e / jnp.sum(e, axis=0, keepdims=True)
    M_ref[0] = lax.dot_general(w, xb, (((0,), (0,)), ((), ())))


def _graph_kernel(M_ref, reh_ref, cW_ref, cb_ref, dW1_ref, db1_ref, dW2_ref,
                  db2_ref, g1W_ref, g1ax_ref, g1ae_ref, g1b_ref, n1w_ref,
                  n1b_ref, n1ms_ref, f1W_ref, f1b_ref, g2W_ref, g2ax_ref,
                  g2ae_ref, g2b_ref, n2w_ref, n2b_ref, n2ms_ref, f2W_ref,
                  f2b_ref, clW_ref, clb_ref, lm_ref, lg_ref):
    M = M_ref[...]
    lm_ref[...] = jnp.dot(M, cW_ref[...]) + cb_ref[...]

    xc = jnp.concatenate([M, reh_ref[...]], axis=0)
    t = _lrelu(jnp.dot(xc, dW1_ref[...]) + db1_ref[...], 0.01)
    h = _lrelu(jnp.dot(t, dW2_ref[...]) + db2_ref[...], 0.01)

    nrm = jnp.sqrt(jnp.sum(h * h, axis=1, keepdims=True))
    hn = h / (nrm + 1e-12)
    sim = lax.dot_general(hn, hn, (((1,), (1,)), ((), ())))

    iota = lax.broadcasted_iota(jnp.int32, (N, N), 1)
    work = sim
    Ps = []
    for _ in range(K):
        m = jnp.max(work, axis=1, keepdims=True)
        ismax = work == m
        idx = jnp.min(jnp.where(ismax, iota, N), axis=1, keepdims=True)
        sel = iota == idx
        Ps.append(sel.astype(jnp.float32))
        work = jnp.where(sel, -1e30, work)
    C = Ps[0] + Ps[1] + Ps[2] + Ps[3]

    ones_col = jnp.ones((N, 1), jnp.float32)
    Dc = _hi_dot_t(C, ones_col)
    D = jnp.where(Dc > 0, 1.0 / jnp.maximum(Dc, 1e-12), 0.0)

    eattr = _hi_dot(C, h) * 0.25

    def hgc(x_in, W, ax, ae, bias):
        xl = jnp.dot(x_in, W)
        he = jnp.dot(eattr, W)
        v = _hi_dot(xl, ax)
        u = _hi_dot(he, ae)
        pre = jnp.concatenate([_hi_dot(Pk, v) for Pk in Ps], axis=1) + u
        a = _lrelu(pre, 0.2)
        amax = jnp.max(a, axis=1, keepdims=True)
        e = jnp.exp(a - amax)
        alpha = e / (jnp.sum(e, axis=1, keepdims=True) + 1e-16)
        Q = (alpha[:, 0:1] * Ps[0] + alpha[:, 1:2] * Ps[1]
             + alpha[:, 2:3] * Ps[2] + alpha[:, 3:4] * Ps[3])
        oute = 0.25 * _hi_dot(Q, xl)
        out = D * _hi_dot_t(Q, oute)
        return out + bias

    def gnorm(hh, w, bb, ms):
        mean = jnp.mean(hh, axis=0, keepdims=True)
        out = hh - ms * mean
        var = jnp.mean(out * out, axis=0, keepdims=True)
        return w * out / jnp.sqrt(var + 1e-5) + bb

    h1 = _lrelu(gnorm(hgc(h, g1W_ref[...], g1ax_ref[...], g1ae_ref[...],
                          g1b_ref[...]), n1w_ref[...], n1b_ref[...],
                      n1ms_ref[...]), 0.01)
    out1 = _lrelu(jnp.dot(h1, f1W_ref[...]) + f1b_ref[...], 0.01)
    h2 = _lrelu(gnorm(hgc(h1, g2W_ref[...], g2ax_ref[...], g2ae_ref[...],
                          g2b_ref[...]), n2w_ref[...], n2b_ref[...],
                      n2ms_ref[...]), 0.01)
    out = out1 + _lrelu(jnp.dot(h2, f2W_ref[...]) + f2b_ref[...], 0.01)
    lg_ref[...] = jnp.dot(out[:B], clW_ref[...]) + clb_ref[...]


def kernel(x, rehearsal, aW1, ab1, aW2, ab2, cW, cb, dW1, db1, dW2, db2,
           g1W, g1att, g1b, n1w, n1b, n1ms, f1W, f1b,
           g2W, g2att, g2b, n2w, n2b, n2ms, f2W, f2b, clW, clb):
    row = lambda v: v.reshape(1, -1)

    M3 = pl.pallas_call(
        _attn_kernel,
        grid=(B,),
        in_specs=[
            pl.BlockSpec((1, NI, F), lambda i: (i, 0, 0)),
            pl.BlockSpec((F, F), lambda i: (0, 0)),
            pl.BlockSpec((1, F), lambda i: (0, 0)),
            pl.BlockSpec((F, 1), lambda i: (0, 0)),
            pl.BlockSpec((1, 1), lambda i: (0, 0)),
        ],
        out_specs=pl.BlockSpec((1, 1, F), lambda i: (i, 0, 0)),
        out_shape=jax.ShapeDtypeStruct((B, 1, F), jnp.float32),
        compiler_params=pltpu.CompilerParams(
            dimension_semantics=("parallel",)),
    )(x, aW1, row(ab1), aW2, ab2.reshape(1, 1))
    M = M3.reshape(B, F)

    g1ax, g1ae = g1att[:F].reshape(F, 1), g1att[F:].reshape(F, 1)
    g2ax, g2ae = g2att[:F].reshape(F, 1), g2att[F:].reshape(F, 1)

    lm, lg = pl.pallas_call(
        _graph_kernel,
        out_shape=[jax.ShapeDtypeStruct((B, NC), jnp.float32),
                   jax.ShapeDtypeStruct((B, NC), jnp.float32)],
    )(M, rehearsal, cW, row(cb), dW1, row(db1), dW2, row(db2),
      g1W, g1ax, g1ae, row(g1b), row(n1w), row(n1b), row(n1ms),
      f1W, row(f1b),
      g2W, g2ax, g2ae, row(g2b), row(n2w), row(n2b), row(n2ms),
      f2W, row(f2b), clW, row(clb))
    return (lm, lg)


# speedup vs baseline: 1.3844x; 1.3844x over previous
import jax
import jax.numpy as jnp
from jax import lax
from jax.experimental import pallas as pl
from jax.experimental.pallas import tpu as pltpu

F = 512
HID = 256
NC = 16
BUF = 256
K = 4
B = 32
NI = 1024
N = B + BUF

def _hi_dot(a, b):
    return lax.dot_general(a, b, (((1,), (0,)), ((), ())),
                           preferred_element_type=jnp.float32)


def _hi_dot_t(a, b):
    return lax.dot_general(a, b, (((0,), (0,)), ((), ())),
                           preferred_element_type=jnp.float32)


def _lrelu(x, slope):
    return jnp.where(x >= 0, x, slope * x)


GB = 4


def _attn_kernel(x_ref, aW1_ref, ab1_ref, aW2_ref, ab2_ref, M_ref):
    xb = x_ref[...]
    H = jnp.maximum(
        lax.dot_general(xb, aW1_ref[...], (((2,), (0,)), ((), ())))
        + ab1_ref[...], 0.0)
    a = (lax.dot_general(H, aW2_ref[...], (((2,), (0,)), ((), ())))
         + ab2_ref[...])
    amax = jnp.max(a, axis=1, keepdims=True)
    e = jnp.exp(a - amax)
    w = e / jnp.sum(e, axis=1, keepdims=True)
    M_ref[...] = lax.dot_general(w, xb, (((1,), (1,)), ((0,), (0,))))


def _graph_kernel(M_ref, reh_ref, cW_ref, cb_ref, dW1_ref, db1_ref, dW2_ref,
                  db2_ref, g1W_ref, g1ax_ref, g1ae_ref, g1b_ref, n1w_ref,
                  n1b_ref, n1ms_ref, f1W_ref, f1b_ref, g2W_ref, g2ax_ref,
                  g2ae_ref, g2b_ref, n2w_ref, n2b_ref, n2ms_ref, f2W_ref,
                  f2b_ref, clW_ref, clb_ref, lm_ref, lg_ref):
    M = M_ref[...]
    lm_ref[...] = jnp.dot(M, cW_ref[...]) + cb_ref[...]

    xc = jnp.concatenate([M, reh_ref[...]], axis=0)
    t = _lrelu(jnp.dot(xc, dW1_ref[...]) + db1_ref[...], 0.01)
    h = _lrelu(jnp.dot(t, dW2_ref[...]) + db2_ref[...], 0.01)

    nrm = jnp.sqrt(jnp.sum(h * h, axis=1, keepdims=True))
    hn = h / (nrm + 1e-12)
    sim = lax.dot_general(hn, hn, (((1,), (1,)), ((), ())))

    iota = lax.broadcasted_iota(jnp.int32, (N, N), 1)
    work = sim
    Ps = []
    for _ in range(K):
        m = jnp.max(work, axis=1, keepdims=True)
        ismax = work == m
        idx = jnp.min(jnp.where(ismax, iota, N), axis=1, keepdims=True)
        sel = iota == idx
        Ps.append(sel.astype(jnp.float32))
        work = jnp.where(sel, -1e30, work)
    C = Ps[0] + Ps[1] + Ps[2] + Ps[3]

    ones_col = jnp.ones((N, 1), jnp.float32)
    Dc = _hi_dot_t(C, ones_col)
    D = jnp.where(Dc > 0, 1.0 / jnp.maximum(Dc, 1e-12), 0.0)

    eattr = _hi_dot(C, h) * 0.25

    def hgc(x_in, W, ax, ae, bias):
        xl = jnp.dot(x_in, W)
        he = jnp.dot(eattr, W)
        v = _hi_dot(xl, ax)
        u = _hi_dot(he, ae)
        pre = jnp.concatenate([_hi_dot(Pk, v) for Pk in Ps], axis=1) + u
        a = _lrelu(pre, 0.2)
        amax = jnp.max(a, axis=1, keepdims=True)
        e = jnp.exp(a - amax)
        alpha = e / (jnp.sum(e, axis=1, keepdims=True) + 1e-16)
        Q = (alpha[:, 0:1] * Ps[0] + alpha[:, 1:2] * Ps[1]
             + alpha[:, 2:3] * Ps[2] + alpha[:, 3:4] * Ps[3])
        oute = 0.25 * _hi_dot(Q, xl)
        out = D * _hi_dot_t(Q, oute)
        return out + bias

    def gnorm(hh, w, bb, ms):
        mean = jnp.mean(hh, axis=0, keepdims=True)
        out = hh - ms * mean
        var = jnp.mean(out * out, axis=0, keepdims=True)
        return w * out / jnp.sqrt(var + 1e-5) + bb

    h1 = _lrelu(gnorm(hgc(h, g1W_ref[...], g1ax_ref[...], g1ae_ref[...],
                          g1b_ref[...]), n1w_ref[...], n1b_ref[...],
                      n1ms_ref[...]), 0.01)
    out1 = _lrelu(jnp.dot(h1, f1W_ref[...]) + f1b_ref[...], 0.01)
    h2 = _lrelu(gnorm(hgc(h1, g2W_ref[...], g2ax_ref[...], g2ae_ref[...],
                          g2b_ref[...]), n2w_ref[...], n2b_ref[...],
                      n2ms_ref[...]), 0.01)
    out = out1 + _lrelu(jnp.dot(h2, f2W_ref[...]) + f2b_ref[...], 0.01)
    lg_ref[...] = jnp.dot(out[:B], clW_ref[...]) + clb_ref[...]


def kernel(x, rehearsal, aW1, ab1, aW2, ab2, cW, cb, dW1, db1, dW2, db2,
           g1W, g1att, g1b, n1w, n1b, n1ms, f1W, f1b,
           g2W, g2att, g2b, n2w, n2b, n2ms, f2W, f2b, clW, clb):
    row = lambda v: v.reshape(1, -1)

    M3 = pl.pallas_call(
        _attn_kernel,
        grid=(B // GB,),
        in_specs=[
            pl.BlockSpec((GB, NI, F), lambda i: (i, 0, 0)),
            pl.BlockSpec((F, F), lambda i: (0, 0)),
            pl.BlockSpec((1, F), lambda i: (0, 0)),
            pl.BlockSpec((F, 1), lambda i: (0, 0)),
            pl.BlockSpec((1, 1), lambda i: (0, 0)),
        ],
        out_specs=pl.BlockSpec((GB, 1, F), lambda i: (i, 0, 0)),
        out_shape=jax.ShapeDtypeStruct((B, 1, F), jnp.float32),
        compiler_params=pltpu.CompilerParams(
            dimension_semantics=("parallel",)),
    )(x, aW1, row(ab1), aW2, ab2.reshape(1, 1))
    M = M3.reshape(B, F)

    g1ax, g1ae = g1att[:F].reshape(F, 1), g1att[F:].reshape(F, 1)
    g2ax, g2ae = g2att[:F].reshape(F, 1), g2att[F:].reshape(F, 1)

    lm, lg = pl.pallas_call(
        _graph_kernel,
        out_shape=[jax.ShapeDtypeStruct((B, NC), jnp.float32),
                   jax.ShapeDtypeStruct((B, NC), jnp.float32)],
    )(M, rehearsal, cW, row(cb), dW1, row(db1), dW2, row(db2),
      g1W, g1ax, g1ae, row(g1b), row(n1w), row(n1b), row(n1ms),
      f1W, row(f1b),
      g2W, g2ax, g2ae, row(g2b), row(n2w), row(n2b), row(n2ms),
      f2W, row(f2b), clW, row(clb))
    return (lm, lg)


# speedup vs baseline: 1.3997x; 1.0110x over previous
import jax
import jax.numpy as jnp
from jax import lax
from jax.experimental import pallas as pl
from jax.experimental.pallas import tpu as pltpu

F = 512
HID = 256
NC = 16
BUF = 256
K = 4
B = 32
NI = 1024
N = B + BUF

def _hi_dot(a, b):
    return lax.dot_general(a, b, (((1,), (0,)), ((), ())),
                           preferred_element_type=jnp.float32)


def _hi_dot_t(a, b):
    return lax.dot_general(a, b, (((0,), (0,)), ((), ())),
                           preferred_element_type=jnp.float32)


def _lrelu(x, slope):
    return jnp.where(x >= 0, x, slope * x)


GB = 8


def _attn_kernel(x_ref, aW1_ref, ab1_ref, aW2_ref, ab2_ref, M_ref):
    xb = x_ref[...]
    H = jnp.maximum(
        lax.dot_general(xb, aW1_ref[...], (((2,), (0,)), ((), ())))
        + ab1_ref[...], 0.0)
    a = (lax.dot_general(H, aW2_ref[...], (((2,), (0,)), ((), ())))
         + ab2_ref[...])
    amax = jnp.max(a, axis=1, keepdims=True)
    e = jnp.exp(a - amax)
    w = e / jnp.sum(e, axis=1, keepdims=True)
    M_ref[...] = lax.dot_general(w, xb, (((1,), (1,)), ((0,), (0,))))


def _graph_kernel(M_ref, reh_ref, cW_ref, cb_ref, dW1_ref, db1_ref, dW2_ref,
                  db2_ref, g1W_ref, g1ax_ref, g1ae_ref, g1b_ref, n1w_ref,
                  n1b_ref, n1ms_ref, f1W_ref, f1b_ref, g2W_ref, g2ax_ref,
                  g2ae_ref, g2b_ref, n2w_ref, n2b_ref, n2ms_ref, f2W_ref,
                  f2b_ref, clW_ref, clb_ref, lm_ref, lg_ref):
    M = M_ref[...]
    lm_ref[...] = jnp.dot(M, cW_ref[...]) + cb_ref[...]

    xc = jnp.concatenate([M, reh_ref[...]], axis=0)
    t = _lrelu(jnp.dot(xc, dW1_ref[...]) + db1_ref[...], 0.01)
    h = _lrelu(jnp.dot(t, dW2_ref[...]) + db2_ref[...], 0.01)

    nrm = jnp.sqrt(jnp.sum(h * h, axis=1, keepdims=True))
    hn = h / (nrm + 1e-12)
    sim = lax.dot_general(hn, hn, (((1,), (1,)), ((), ())))

    iota = lax.broadcasted_iota(jnp.int32, (N, N), 1)
    work = sim
    Ps = []
    for _ in range(K):
        m = jnp.max(work, axis=1, keepdims=True)
        ismax = work == m
        idx = jnp.min(jnp.where(ismax, iota, N), axis=1, keepdims=True)
        sel = iota == idx
        Ps.append(sel.astype(jnp.float32))
        work = jnp.where(sel, -1e30, work)
    C = Ps[0] + Ps[1] + Ps[2] + Ps[3]

    ones_col = jnp.ones((N, 1), jnp.float32)
    Dc = _hi_dot_t(C, ones_col)
    D = jnp.where(Dc > 0, 1.0 / jnp.maximum(Dc, 1e-12), 0.0)

    eattr = _hi_dot(C, h) * 0.25

    def hgc(x_in, W, ax, ae, bias):
        xl = jnp.dot(x_in, W)
        he = jnp.dot(eattr, W)
        v = _hi_dot(xl, ax)
        u = _hi_dot(he, ae)
        pre = jnp.concatenate([_hi_dot(Pk, v) for Pk in Ps], axis=1) + u
        a = _lrelu(pre, 0.2)
        amax = jnp.max(a, axis=1, keepdims=True)
        e = jnp.exp(a - amax)
        alpha = e / (jnp.sum(e, axis=1, keepdims=True) + 1e-16)
        Q = (alpha[:, 0:1] * Ps[0] + alpha[:, 1:2] * Ps[1]
             + alpha[:, 2:3] * Ps[2] + alpha[:, 3:4] * Ps[3])
        oute = 0.25 * _hi_dot(Q, xl)
        out = D * _hi_dot_t(Q, oute)
        return out + bias

    def gnorm(hh, w, bb, ms):
        mean = jnp.mean(hh, axis=0, keepdims=True)
        out = hh - ms * mean
        var = jnp.mean(out * out, axis=0, keepdims=True)
        return w * out / jnp.sqrt(var + 1e-5) + bb

    h1 = _lrelu(gnorm(hgc(h, g1W_ref[...], g1ax_ref[...], g1ae_ref[...],
                          g1b_ref[...]), n1w_ref[...], n1b_ref[...],
                      n1ms_ref[...]), 0.01)
    out1 = _lrelu(jnp.dot(h1, f1W_ref[...]) + f1b_ref[...], 0.01)
    h2 = _lrelu(gnorm(hgc(h1, g2W_ref[...], g2ax_ref[...], g2ae_ref[...],
                          g2b_ref[...]), n2w_ref[...], n2b_ref[...],
                      n2ms_ref[...]), 0.01)
    out = out1 + _lrelu(jnp.dot(h2, f2W_ref[...]) + f2b_ref[...], 0.01)
    lg_ref[...] = jnp.dot(out[:B], clW_ref[...]) + clb_ref[...]


def kernel(x, rehearsal, aW1, ab1, aW2, ab2, cW, cb, dW1, db1, dW2, db2,
           g1W, g1att, g1b, n1w, n1b, n1ms, f1W, f1b,
           g2W, g2att, g2b, n2w, n2b, n2ms, f2W, f2b, clW, clb):
    row = lambda v: v.reshape(1, -1)

    M3 = pl.pallas_call(
        _attn_kernel,
        grid=(B // GB,),
        in_specs=[
            pl.BlockSpec((GB, NI, F), lambda i: (i, 0, 0)),
            pl.BlockSpec((F, F), lambda i: (0, 0)),
            pl.BlockSpec((1, F), lambda i: (0, 0)),
            pl.BlockSpec((F, 1), lambda i: (0, 0)),
            pl.BlockSpec((1, 1), lambda i: (0, 0)),
        ],
        out_specs=pl.BlockSpec((GB, 1, F), lambda i: (i, 0, 0)),
        out_shape=jax.ShapeDtypeStruct((B, 1, F), jnp.float32),
        compiler_params=pltpu.CompilerParams(
            dimension_semantics=("parallel",)),
    )(x, aW1, row(ab1), aW2, ab2.reshape(1, 1))
    M = M3.reshape(B, F)

    g1ax, g1ae = g1att[:F].reshape(F, 1), g1att[F:].reshape(F, 1)
    g2ax, g2ae = g2att[:F].reshape(F, 1), g2att[F:].reshape(F, 1)

    lm, lg = pl.pallas_call(
        _graph_kernel,
        out_shape=[jax.ShapeDtypeStruct((B, NC), jnp.float32),
                   jax.ShapeDtypeStruct((B, NC), jnp.float32)],
    )(M, rehearsal, cW, row(cb), dW1, row(db1), dW2, row(db2),
      g1W, g1ax, g1ae, row(g1b), row(n1w), row(n1b), row(n1ms),
      f1W, row(f1b),
      g2W, g2ax, g2ae, row(g2b), row(n2w), row(n2b), row(n2ms),
      f2W, row(f2b), clW, row(clb))
    return (lm, lg)
